# Initial kernel scaffold; baseline (speedup 1.0000x reference)
import jax
import jax.numpy as jnp
from jax.experimental import pallas as pl


def kernel(nbr_fea, angle_fea, angle_nbr_idx, crystal_edge_idx, crystal_angle_index, W_full, cn_gamma, cn_beta, W_mask, ln_core_g, ln_core_b, ln2_g, ln2_b, W1a, b1a, W2a, b2a, W1b, b1b, W2b, b2b):
    raise NotImplementedError("write your pallas kernel here")


# speedup vs baseline: 4.4861x; 4.4861x over previous
import jax
import jax.numpy as jnp
from jax.experimental import pallas as pl
from jax.experimental.pallas import tpu as pltpu

E = 160000
A = 320000
F = 128
FA = 16
NC = 256
EPS = 1e-5
INV_SQRT_2 = 1.0 / (2.0 ** 0.5)

BA = 3200
BE = 3200


def _k1(taf_ref, idx_ref, w_ref, tg_ref, sums_ref, sumsq_ref, cnt_ref,
        acc_s, acc_q, acc_c):
    i = pl.program_id(0)
    tg = jnp.dot(taf_ref[...], w_ref[...], preferred_element_type=jnp.float32)
    tg_ref[...] = tg
    idx = idx_ref[...]
    onehot = (idx == jax.lax.broadcasted_iota(jnp.int32, (BA, NC), 1)
              ).astype(jnp.float32)

    @pl.when(i == 0)
    def _init():
        acc_s[...] = jnp.zeros_like(acc_s)
        acc_q[...] = jnp.zeros_like(acc_q)
        acc_c[...] = jnp.zeros_like(acc_c)

    acc_s[...] += jnp.dot(onehot.T, tg, preferred_element_type=jnp.float32)
    acc_q[...] += jnp.dot(onehot.T, tg * tg,
                          preferred_element_type=jnp.float32)
    acc_c[...] += jnp.sum(onehot, axis=0, keepdims=True)

    @pl.when(i == pl.num_programs(0) - 1)
    def _flush():
        sums_ref[...] = acc_s[...]
        sumsq_ref[...] = acc_q[...]
        cnt_ref[...] = acc_c[...]


def _k2(tg_ref, idx_ref, sums_ref, sumsq_ref, cnt_ref, cng_ref, cnb_ref,
        wm_ref, lng_ref, lnb_ref, out_ref):
    cnt = jnp.maximum(cnt_ref[...], 1.0)
    mean = sums_ref[...] / cnt.T
    var = sumsq_ref[...] / cnt.T - mean * mean
    idx = idx_ref[...]
    onehot = (idx == jax.lax.broadcasted_iota(jnp.int32, (BA, NC), 1)
              ).astype(jnp.float32)
    mean_g = jnp.dot(onehot, mean, preferred_element_type=jnp.float32)
    var_g = jnp.dot(onehot, var, preferred_element_type=jnp.float32)
    tg = (tg_ref[...] - mean_g) * jax.lax.rsqrt(var_g + EPS) \
        * cng_ref[...] + cnb_ref[...]
    core = tg[:, :F]
    filt = tg[:, F:]
    mu = jnp.mean(core, axis=-1, keepdims=True)
    v = jnp.mean((core - mu) ** 2, axis=-1, keepdims=True)
    core = (core - mu) * jax.lax.rsqrt(v + EPS) * lng_ref[...] + lnb_ref[...]
    core = core * jax.nn.sigmoid(core)
    gate = jax.nn.sigmoid(jnp.dot(filt, wm_ref[...],
                                  preferred_element_type=jnp.float32))
    out_ref[...] = gate * core


def _k3(ssum_ref, scnt_ref, nbr_ref, g2_ref, b2_ref,
        w1a_ref, b1a_ref, w2a_ref, b2a_ref,
        w1b_ref, b1b_ref, w2b_ref, b2b_ref, out_ref):
    x = ssum_ref[...] / jnp.maximum(scnt_ref[...], 1.0)
    mu = jnp.mean(x, axis=-1, keepdims=True)
    v = jnp.mean((x - mu) ** 2, axis=-1, keepdims=True)
    x = (x - mu) * jax.lax.rsqrt(v + EPS) * g2_ref[...] + b2_ref[...]
    h = jnp.dot(x, w1a_ref[...], preferred_element_type=jnp.float32) \
        + b1a_ref[...]
    h = h * jax.nn.sigmoid(h)
    h = jnp.dot(h, w2a_ref[...], preferred_element_type=jnp.float32) \
        + b2a_ref[...]
    h = h * jax.nn.sigmoid(h)
    x = x + h
    h = jnp.dot(x, w1b_ref[...], preferred_element_type=jnp.float32) \
        + b1b_ref[...]
    h = h * jax.nn.sigmoid(h)
    h = jnp.dot(h, w2b_ref[...], preferred_element_type=jnp.float32) \
        + b2b_ref[...]
    h = h * jax.nn.sigmoid(h)
    x = x + h
    out_ref[...] = INV_SQRT_2 * (nbr_ref[...] + x)


def kernel(nbr_fea, angle_fea, angle_nbr_idx, crystal_edge_idx,
           crystal_angle_index, W_full, cn_gamma, cn_beta, W_mask,
           ln_core_g, ln_core_b, ln2_g, ln2_b, W1a, b1a, W2a, b2a,
           W1b, b1b, W2b, b2b):
    source_idx = angle_nbr_idx[:, 0]
    angle_nbr_fea = nbr_fea[angle_nbr_idx].reshape(-1, 2 * F)
    taf = jnp.concatenate([angle_fea, angle_nbr_fea], axis=1)
    cai = crystal_angle_index.astype(jnp.int32).reshape(A, 1)

    grid1 = A // BA
    tg, sums, sumsq, cnt = pl.pallas_call(
        _k1,
        grid=(grid1,),
        in_specs=[
            pl.BlockSpec((BA, 2 * F + FA), lambda i: (i, 0)),
            pl.BlockSpec((BA, 1), lambda i: (i, 0)),
            pl.BlockSpec((2 * F + FA, 2 * F), lambda i: (0, 0)),
        ],
        out_specs=[
            pl.BlockSpec((BA, 2 * F), lambda i: (i, 0)),
            pl.BlockSpec((NC, 2 * F), lambda i: (0, 0)),
            pl.BlockSpec((NC, 2 * F), lambda i: (0, 0)),
            pl.BlockSpec((1, NC), lambda i: (0, 0)),
        ],
        out_shape=[
            jax.ShapeDtypeStruct((A, 2 * F), jnp.float32),
            jax.ShapeDtypeStruct((NC, 2 * F), jnp.float32),
            jax.ShapeDtypeStruct((NC, 2 * F), jnp.float32),
            jax.ShapeDtypeStruct((1, NC), jnp.float32),
        ],
        scratch_shapes=[
            pltpu.VMEM((NC, 2 * F), jnp.float32),
            pltpu.VMEM((NC, 2 * F), jnp.float32),
            pltpu.VMEM((1, NC), jnp.float32),
        ],
    )(taf, cai, W_full)

    gated = pl.pallas_call(
        _k2,
        grid=(grid1,),
        in_specs=[
            pl.BlockSpec((BA, 2 * F), lambda i: (i, 0)),
            pl.BlockSpec((BA, 1), lambda i: (i, 0)),
            pl.BlockSpec((NC, 2 * F), lambda i: (0, 0)),
            pl.BlockSpec((NC, 2 * F), lambda i: (0, 0)),
            pl.BlockSpec((1, NC), lambda i: (0, 0)),
            pl.BlockSpec((1, 2 * F), lambda i: (0, 0)),
            pl.BlockSpec((1, 2 * F), lambda i: (0, 0)),
            pl.BlockSpec((F, 1), lambda i: (0, 0)),
            pl.BlockSpec((1, F), lambda i: (0, 0)),
            pl.BlockSpec((1, F), lambda i: (0, 0)),
        ],
        out_specs=pl.BlockSpec((BA, F), lambda i: (i, 0)),
        out_shape=jax.ShapeDtypeStruct((A, F), jnp.float32),
    )(tg, cai, sums, sumsq, cnt, cn_gamma.reshape(1, -1),
      cn_beta.reshape(1, -1), W_mask, ln_core_g.reshape(1, -1),
      ln_core_b.reshape(1, -1))

    ssum = jax.ops.segment_sum(gated, source_idx, num_segments=E)
    scnt = jax.ops.segment_sum(jnp.ones((A, 1), jnp.float32), source_idx,
                               num_segments=E)

    grid3 = E // BE
    out = pl.pallas_call(
        _k3,
        grid=(grid3,),
        in_specs=[
            pl.BlockSpec((BE, F), lambda i: (i, 0)),
            pl.BlockSpec((BE, 1), lambda i: (i, 0)),
            pl.BlockSpec((BE, F), lambda i: (i, 0)),
            pl.BlockSpec((1, F), lambda i: (0, 0)),
            pl.BlockSpec((1, F), lambda i: (0, 0)),
            pl.BlockSpec((F, F // 2), lambda i: (0, 0)),
            pl.BlockSpec((1, F // 2), lambda i: (0, 0)),
            pl.BlockSpec((F // 2, F), lambda i: (0, 0)),
            pl.BlockSpec((1, F), lambda i: (0, 0)),
            pl.BlockSpec((F, F // 2), lambda i: (0, 0)),
            pl.BlockSpec((1, F // 2), lambda i: (0, 0)),
            pl.BlockSpec((F // 2, F), lambda i: (0, 0)),
            pl.BlockSpec((1, F), lambda i: (0, 0)),
        ],
        out_specs=pl.BlockSpec((BE, F), lambda i: (i, 0)),
        out_shape=jax.ShapeDtypeStruct((E, F), jnp.float32),
    )(ssum, scnt, nbr_fea, ln2_g.reshape(1, -1), ln2_b.reshape(1, -1),
      W1a, b1a.reshape(1, -1), W2a, b2a.reshape(1, -1),
      W1b, b1b.reshape(1, -1), W2b, b2b.reshape(1, -1))

    return out


# speedup vs baseline: 4.5959x; 1.0245x over previous
import jax
import jax.numpy as jnp
from jax.experimental import pallas as pl
from jax.experimental.pallas import tpu as pltpu

E = 160000
A = 320000
F = 128
FA = 16
NC = 256
EPS = 1e-5
INV_SQRT_2 = 1.0 / (2.0 ** 0.5)

BA = 6400
BE = 6400


def _k1(taf_ref, idx_ref, w_ref, tg_ref, sums_ref, sumsq_ref, cnt_ref,
        acc_s, acc_q, acc_c):
    i = pl.program_id(0)
    tg = jnp.dot(taf_ref[...], w_ref[...], preferred_element_type=jnp.float32)
    tg_ref[...] = tg
    idx = idx_ref[...]
    onehot = (idx == jax.lax.broadcasted_iota(jnp.int32, (BA, NC), 1)
              ).astype(jnp.float32)

    @pl.when(i == 0)
    def _init():
        acc_s[...] = jnp.zeros_like(acc_s)
        acc_q[...] = jnp.zeros_like(acc_q)
        acc_c[...] = jnp.zeros_like(acc_c)

    acc_s[...] += jnp.dot(onehot.T, tg, preferred_element_type=jnp.float32)
    acc_q[...] += jnp.dot(onehot.T, tg * tg,
                          preferred_element_type=jnp.float32)
    acc_c[...] += jnp.sum(onehot, axis=0, keepdims=True)

    @pl.when(i == pl.num_programs(0) - 1)
    def _flush():
        sums_ref[...] = acc_s[...]
        sumsq_ref[...] = acc_q[...]
        cnt_ref[...] = acc_c[...]


def _k2(tg_ref, idx_ref, sums_ref, sumsq_ref, cnt_ref, cng_ref, cnb_ref,
        wm_ref, lng_ref, lnb_ref, out_ref):
    cnt = jnp.maximum(cnt_ref[...], 1.0)
    mean = sums_ref[...] / cnt.T
    var = sumsq_ref[...] / cnt.T - mean * mean
    idx = idx_ref[...]
    onehot = (idx == jax.lax.broadcasted_iota(jnp.int32, (BA, NC), 1)
              ).astype(jnp.float32)
    mean_g = jnp.dot(onehot, mean, preferred_element_type=jnp.float32)
    var_g = jnp.dot(onehot, var, preferred_element_type=jnp.float32)
    tg = (tg_ref[...] - mean_g) * jax.lax.rsqrt(var_g + EPS) \
        * cng_ref[...] + cnb_ref[...]
    core = tg[:, :F]
    filt = tg[:, F:]
    mu = jnp.mean(core, axis=-1, keepdims=True)
    v = jnp.mean((core - mu) ** 2, axis=-1, keepdims=True)
    core = (core - mu) * jax.lax.rsqrt(v + EPS) * lng_ref[...] + lnb_ref[...]
    core = core * jax.nn.sigmoid(core)
    gate = jax.nn.sigmoid(jnp.dot(filt, wm_ref[...],
                                  preferred_element_type=jnp.float32))
    out_ref[...] = gate * core


def _k3(ssum_ref, scnt_ref, nbr_ref, g2_ref, b2_ref,
        w1a_ref, b1a_ref, w2a_ref, b2a_ref,
        w1b_ref, b1b_ref, w2b_ref, b2b_ref, out_ref):
    x = ssum_ref[...] / jnp.maximum(scnt_ref[...], 1.0)
    mu = jnp.mean(x, axis=-1, keepdims=True)
    v = jnp.mean((x - mu) ** 2, axis=-1, keepdims=True)
    x = (x - mu) * jax.lax.rsqrt(v + EPS) * g2_ref[...] + b2_ref[...]
    h = jnp.dot(x, w1a_ref[...], preferred_element_type=jnp.float32) \
        + b1a_ref[...]
    h = h * jax.nn.sigmoid(h)
    h = jnp.dot(h, w2a_ref[...], preferred_element_type=jnp.float32) \
        + b2a_ref[...]
    h = h * jax.nn.sigmoid(h)
    x = x + h
    h = jnp.dot(x, w1b_ref[...], preferred_element_type=jnp.float32) \
        + b1b_ref[...]
    h = h * jax.nn.sigmoid(h)
    h = jnp.dot(h, w2b_ref[...], preferred_element_type=jnp.float32) \
        + b2b_ref[...]
    h = h * jax.nn.sigmoid(h)
    x = x + h
    out_ref[...] = INV_SQRT_2 * (nbr_ref[...] + x)


def kernel(nbr_fea, angle_fea, angle_nbr_idx, crystal_edge_idx,
           crystal_angle_index, W_full, cn_gamma, cn_beta, W_mask,
           ln_core_g, ln_core_b, ln2_g, ln2_b, W1a, b1a, W2a, b2a,
           W1b, b1b, W2b, b2b):
    source_idx = angle_nbr_idx[:, 0]
    angle_nbr_fea = nbr_fea[angle_nbr_idx].reshape(-1, 2 * F)
    taf = jnp.concatenate([angle_fea, angle_nbr_fea], axis=1)
    cai = crystal_angle_index.astype(jnp.int32).reshape(A, 1)

    grid1 = A // BA
    tg, sums, sumsq, cnt = pl.pallas_call(
        _k1,
        grid=(grid1,),
        in_specs=[
            pl.BlockSpec((BA, 2 * F + FA), lambda i: (i, 0)),
            pl.BlockSpec((BA, 1), lambda i: (i, 0)),
            pl.BlockSpec((2 * F + FA, 2 * F), lambda i: (0, 0)),
        ],
        out_specs=[
            pl.BlockSpec((BA, 2 * F), lambda i: (i, 0)),
            pl.BlockSpec((NC, 2 * F), lambda i: (0, 0)),
            pl.BlockSpec((NC, 2 * F), lambda i: (0, 0)),
            pl.BlockSpec((1, NC), lambda i: (0, 0)),
        ],
        out_shape=[
            jax.ShapeDtypeStruct((A, 2 * F), jnp.float32),
            jax.ShapeDtypeStruct((NC, 2 * F), jnp.float32),
            jax.ShapeDtypeStruct((NC, 2 * F), jnp.float32),
            jax.ShapeDtypeStruct((1, NC), jnp.float32),
        ],
        scratch_shapes=[
            pltpu.VMEM((NC, 2 * F), jnp.float32),
            pltpu.VMEM((NC, 2 * F), jnp.float32),
            pltpu.VMEM((1, NC), jnp.float32),
        ],
    )(taf, cai, W_full)

    gated = pl.pallas_call(
        _k2,
        grid=(grid1,),
        in_specs=[
            pl.BlockSpec((BA, 2 * F), lambda i: (i, 0)),
            pl.BlockSpec((BA, 1), lambda i: (i, 0)),
            pl.BlockSpec((NC, 2 * F), lambda i: (0, 0)),
            pl.BlockSpec((NC, 2 * F), lambda i: (0, 0)),
            pl.BlockSpec((1, NC), lambda i: (0, 0)),
            pl.BlockSpec((1, 2 * F), lambda i: (0, 0)),
            pl.BlockSpec((1, 2 * F), lambda i: (0, 0)),
            pl.BlockSpec((F, 1), lambda i: (0, 0)),
            pl.BlockSpec((1, F), lambda i: (0, 0)),
            pl.BlockSpec((1, F), lambda i: (0, 0)),
        ],
        out_specs=pl.BlockSpec((BA, F), lambda i: (i, 0)),
        out_shape=jax.ShapeDtypeStruct((A, F), jnp.float32),
    )(tg, cai, sums, sumsq, cnt, cn_gamma.reshape(1, -1),
      cn_beta.reshape(1, -1), W_mask, ln_core_g.reshape(1, -1),
      ln_core_b.reshape(1, -1))

    ssum = jax.ops.segment_sum(gated, source_idx, num_segments=E)
    scnt = jax.ops.segment_sum(jnp.ones((A, 1), jnp.float32), source_idx,
                               num_segments=E)

    grid3 = E // BE
    out = pl.pallas_call(
        _k3,
        grid=(grid3,),
        in_specs=[
            pl.BlockSpec((BE, F), lambda i: (i, 0)),
            pl.BlockSpec((BE, 1), lambda i: (i, 0)),
            pl.BlockSpec((BE, F), lambda i: (i, 0)),
            pl.BlockSpec((1, F), lambda i: (0, 0)),
            pl.BlockSpec((1, F), lambda i: (0, 0)),
            pl.BlockSpec((F, F // 2), lambda i: (0, 0)),
            pl.BlockSpec((1, F // 2), lambda i: (0, 0)),
            pl.BlockSpec((F // 2, F), lambda i: (0, 0)),
            pl.BlockSpec((1, F), lambda i: (0, 0)),
            pl.BlockSpec((F, F // 2), lambda i: (0, 0)),
            pl.BlockSpec((1, F // 2), lambda i: (0, 0)),
            pl.BlockSpec((F // 2, F), lambda i: (0, 0)),
            pl.BlockSpec((1, F), lambda i: (0, 0)),
        ],
        out_specs=pl.BlockSpec((BE, F), lambda i: (i, 0)),
        out_shape=jax.ShapeDtypeStruct((E, F), jnp.float32),
    )(ssum, scnt, nbr_fea, ln2_g.reshape(1, -1), ln2_b.reshape(1, -1),
      W1a, b1a.reshape(1, -1), W2a, b2a.reshape(1, -1),
      W1b, b1b.reshape(1, -1), W2b, b2b.reshape(1, -1))

    return out
